# baseline (speedup 1.0000x reference)
import functools

import jax
import jax.numpy as jnp
import numpy as np
from jax.experimental import pallas as pl
from jax.experimental.pallas import tpu as pltpu

_A = 2
_K_FRAC = 0.05

_INT_MIN = np.int32(-2147483648)
_INT_LOW31 = np.int32(2147483647)


def _f32_key(u):
    return u ^ ((u >> 31) & _INT_LOW31)


def _encode_kernel(x_ref, wt_ref, be_ref, out_ref):
    out_ref[...] = jax.lax.dot_general(
        x_ref[...], wt_ref[...], (((1,), (0,)), ((), ())),
        preferred_element_type=jnp.float32) + be_ref[...]


def _select_decode_kernel(a1_ref, w_ref, bd_ref, out_ref, a1s_ref,
                          *, k_count, bk):
    kb = pl.program_id(1)

    @pl.when(kb == 0)
    def _select():
        a1 = a1_ref[...]
        rb = a1.astype(jnp.bfloat16)
        kf = jnp.float32(k_count)
        bm, n = a1.shape
        one_b = np.array(1, jnp.bfloat16)
        zero_b = np.array(0, jnp.bfloat16)

        def count_bf(tb):
            m = jnp.where(rb >= tb, one_b, zero_b)
            parts = [m[:, i * 128:(i + 1) * 128] for i in range(n // 128)]
            while len(parts) > 1:
                parts = [parts[i] + parts[i + 1]
                         for i in range(0, len(parts), 2)]
            return jnp.sum(parts[0].astype(jnp.float32), axis=1,
                           keepdims=True)

        def body_a(i, cand_u):
            trial_u = cand_u | jnp.left_shift(jnp.int32(1), 15 - i)
            trial_s = trial_u - np.int32(32768)
            pat16 = trial_s ^ ((trial_s >> 15) & np.int32(0x7FFF))
            tb32 = jax.lax.bitcast_convert_type(
                jnp.left_shift(pat16, 16), jnp.float32)
            cnt = count_bf(tb32.astype(jnp.bfloat16))
            return jnp.where(cnt >= kf, trial_u, cand_u)

        cand_u = jax.lax.fori_loop(
            0, 16, body_a, jnp.zeros((bm, 1), jnp.int32))

        cs = cand_u - np.int32(32768)
        k32 = _f32_key(jnp.left_shift(cs ^ ((cs >> 15) & np.int32(0x7FFF)),
                                      16))

        half = np.int32(32769)

        def body_b(i, lo_hi):
            lo, hi = lo_hi
            mid = lo + ((hi - lo) >> 1)
            tf = jax.lax.bitcast_convert_type(_f32_key(mid), jnp.float32)
            cnt = jnp.sum((a1 >= tf).astype(jnp.float32), axis=1,
                          keepdims=True)
            ok = cnt >= kf
            return jnp.where(ok, mid, lo), jnp.where(ok, hi, mid)

        lo, _ = jax.lax.fori_loop(0, 9, body_b, (k32 - half, k32 + half))
        thr = jax.lax.bitcast_convert_type(_f32_key(lo), jnp.float32)

        a1s_ref[...] = jnp.where(a1 >= thr, a1, 0.0).astype(jnp.bfloat16)

    part = jax.lax.dot_general(
        a1s_ref[:, pl.ds(kb * bk, bk)], w_ref[...], (((1,), (0,)), ((), ())),
        preferred_element_type=jnp.float32)

    @pl.when(kb == 0)
    def _init_out():
        out_ref[...] = bd_ref[...] + part

    @pl.when(kb > 0)
    def _acc_out():
        out_ref[...] += part


def kernel(x, W_enc, b_enc, b_dec, epoch):
    if x.ndim == 1:
        x = x[None, :]
    batch, in_dim = x.shape
    bn_dim = W_enc.shape[0]
    k_count = max(1, int(bn_dim * _A * _K_FRAC))

    x_bf = x.astype(jnp.bfloat16)
    w_bf = W_enc.astype(jnp.bfloat16)
    wt_bf = w_bf.T
    be2 = b_enc.reshape(1, -1).astype(jnp.float32)
    bd2 = b_dec.reshape(1, -1).astype(jnp.float32)

    bm_e, bn_e = min(512, batch), min(1024, bn_dim)
    a1 = pl.pallas_call(
        _encode_kernel,
        grid=(bn_dim // bn_e, batch // bm_e),
        in_specs=[
            pl.BlockSpec((bm_e, in_dim), lambda bn, bm: (bm, 0)),
            pl.BlockSpec((in_dim, bn_e), lambda bn, bm: (0, bn)),
            pl.BlockSpec((1, bn_e), lambda bn, bm: (0, bn)),
        ],
        out_specs=pl.BlockSpec((bm_e, bn_e), lambda bn, bm: (bm, bn)),
        out_shape=jax.ShapeDtypeStruct((batch, bn_dim), jnp.float32),
    )(x_bf, wt_bf, be2)

    bm_d, bk_d = min(256, batch), min(2048, bn_dim)
    n_kb = bn_dim // bk_d
    z2 = pl.pallas_call(
        functools.partial(_select_decode_kernel, k_count=k_count, bk=bk_d),
        grid=(batch // bm_d, n_kb),
        in_specs=[
            pl.BlockSpec((bm_d, bn_dim), lambda bm, kb: (bm, 0)),
            pl.BlockSpec((bk_d, in_dim), lambda bm, kb: (kb, 0)),
            pl.BlockSpec((1, in_dim), lambda bm, kb: (0, 0)),
        ],
        out_specs=pl.BlockSpec((bm_d, in_dim), lambda bm, kb: (bm, 0)),
        out_shape=jax.ShapeDtypeStruct((batch, in_dim), jnp.float32),
        scratch_shapes=[pltpu.VMEM((bm_d, bn_dim), jnp.bfloat16)],
    )(a1, w_bf, bd2)

    return z2


# speedup vs baseline: 1.1100x; 1.1100x over previous
import functools

import jax
import jax.numpy as jnp
import numpy as np
from jax.experimental import pallas as pl
from jax.experimental.pallas import tpu as pltpu

_A = 2
_K_FRAC = 0.05

_INT_MIN = np.int32(-2147483648)
_INT_LOW31 = np.int32(2147483647)


def _f32_key(u):
    return u ^ ((u >> 31) & _INT_LOW31)


def _encode_kernel(x_ref, wt_ref, be_ref, out_ref):
    out_ref[...] = jax.lax.dot_general(
        x_ref[...], wt_ref[...], (((1,), (0,)), ((), ())),
        preferred_element_type=jnp.float32) + be_ref[...]


def _select_kernel(a1_ref, a1s_ref, *, k_count):
    a1 = a1_ref[...]
    rb = a1.astype(jnp.bfloat16)
    kf = jnp.float32(k_count)
    bm, n = a1.shape
    one_b = np.array(1, jnp.bfloat16)
    zero_b = np.array(0, jnp.bfloat16)

    def count_bf(tb):
        m = jnp.where(rb >= tb, one_b, zero_b)
        parts = [m[:, i * 128:(i + 1) * 128] for i in range(n // 128)]
        while len(parts) > 1:
            parts = [parts[i] + parts[i + 1]
                     for i in range(0, len(parts), 2)]
        return jnp.sum(parts[0].astype(jnp.float32), axis=1, keepdims=True)

    def body_a(i, cand_u):
        trial_u = cand_u | jnp.left_shift(jnp.int32(1), 15 - i)
        trial_s = trial_u - np.int32(32768)
        pat16 = trial_s ^ ((trial_s >> 15) & np.int32(0x7FFF))
        tb32 = jax.lax.bitcast_convert_type(
            jnp.left_shift(pat16, 16), jnp.float32)
        cnt = count_bf(tb32.astype(jnp.bfloat16))
        return jnp.where(cnt >= kf, trial_u, cand_u)

    cand_u = jax.lax.fori_loop(
        0, 16, body_a, jnp.zeros((bm, 1), jnp.int32))

    cs = cand_u - np.int32(32768)
    k32 = _f32_key(jnp.left_shift(cs ^ ((cs >> 15) & np.int32(0x7FFF)), 16))

    half = np.int32(32769)

    def body_b(i, lo_hi):
        lo, hi = lo_hi
        mid = lo + ((hi - lo) >> 1)
        tf = jax.lax.bitcast_convert_type(_f32_key(mid), jnp.float32)
        cnt = jnp.sum((a1 >= tf).astype(jnp.float32), axis=1, keepdims=True)
        ok = cnt >= kf
        return jnp.where(ok, mid, lo), jnp.where(ok, hi, mid)

    lo, _ = jax.lax.fori_loop(0, 9, body_b, (k32 - half, k32 + half))
    thr = jax.lax.bitcast_convert_type(_f32_key(lo), jnp.float32)

    a1s_ref[...] = jnp.where(a1 >= thr, a1, 0.0).astype(jnp.bfloat16)


def _decode_kernel(a1s_ref, w_ref, bd_ref, out_ref):
    kb = pl.program_id(1)
    part = jax.lax.dot_general(
        a1s_ref[...], w_ref[...], (((1,), (0,)), ((), ())),
        preferred_element_type=jnp.float32)

    @pl.when(kb == 0)
    def _init_out():
        out_ref[...] = bd_ref[...] + part

    @pl.when(kb > 0)
    def _acc_out():
        out_ref[...] += part


def kernel(x, W_enc, b_enc, b_dec, epoch):
    if x.ndim == 1:
        x = x[None, :]
    batch, in_dim = x.shape
    bn_dim = W_enc.shape[0]
    k_count = max(1, int(bn_dim * _A * _K_FRAC))

    x_bf = x.astype(jnp.bfloat16)
    w_bf = W_enc.astype(jnp.bfloat16)
    wt_bf = w_bf.T
    be2 = b_enc.reshape(1, -1).astype(jnp.float32)
    bd2 = b_dec.reshape(1, -1).astype(jnp.float32)

    bm_e, bn_e = min(512, batch), min(1024, bn_dim)
    a1 = pl.pallas_call(
        _encode_kernel,
        grid=(bn_dim // bn_e, batch // bm_e),
        in_specs=[
            pl.BlockSpec((bm_e, in_dim), lambda bn, bm: (bm, 0)),
            pl.BlockSpec((in_dim, bn_e), lambda bn, bm: (0, bn)),
            pl.BlockSpec((1, bn_e), lambda bn, bm: (0, bn)),
        ],
        out_specs=pl.BlockSpec((bm_e, bn_e), lambda bn, bm: (bm, bn)),
        out_shape=jax.ShapeDtypeStruct((batch, bn_dim), jnp.float32),
    )(x_bf, wt_bf, be2)

    bm_t = min(256, batch)
    a1s = pl.pallas_call(
        functools.partial(_select_kernel, k_count=k_count),
        grid=(batch // bm_t,),
        in_specs=[pl.BlockSpec((bm_t, bn_dim), lambda bm: (bm, 0))],
        out_specs=pl.BlockSpec((bm_t, bn_dim), lambda bm: (bm, 0)),
        out_shape=jax.ShapeDtypeStruct((batch, bn_dim), jnp.bfloat16),
    )(a1)

    bm_d, bk_d = min(1024, batch), min(2048, bn_dim)
    z2 = pl.pallas_call(
        _decode_kernel,
        grid=(batch // bm_d, bn_dim // bk_d),
        in_specs=[
            pl.BlockSpec((bm_d, bk_d), lambda bm, kb: (bm, kb)),
            pl.BlockSpec((bk_d, in_dim), lambda bm, kb: (kb, 0)),
            pl.BlockSpec((1, in_dim), lambda bm, kb: (0, 0)),
        ],
        out_specs=pl.BlockSpec((bm_d, in_dim), lambda bm, kb: (bm, 0)),
        out_shape=jax.ShapeDtypeStruct((batch, in_dim), jnp.float32),
    )(a1s, w_bf, bd2)

    return z2


# speedup vs baseline: 1.1257x; 1.0141x over previous
import functools

import jax
import jax.numpy as jnp
import numpy as np
from jax.experimental import pallas as pl
from jax.experimental.pallas import tpu as pltpu

_A = 2
_K_FRAC = 0.05

_INT_MIN = np.int32(-2147483648)
_INT_LOW31 = np.int32(2147483647)


def _f32_key(u):
    return u ^ ((u >> 31) & _INT_LOW31)


def _encode_kernel(x_ref, wt_ref, be_ref, out_ref):
    out_ref[...] = jax.lax.dot_general(
        x_ref[...], wt_ref[...], (((1,), (0,)), ((), ())),
        preferred_element_type=jnp.float32) + be_ref[...]


def _select_kernel(a1_ref, a1s_ref, *, k_count):
    a1 = a1_ref[...]
    rb = a1.astype(jnp.bfloat16)
    kf = jnp.float32(k_count)
    bm, n = a1.shape
    one_b = np.array(1, jnp.bfloat16)
    zero_b = np.array(0, jnp.bfloat16)

    def count_bf(tb):
        m = jnp.where(rb >= tb, one_b, zero_b)
        parts = [m[:, i * 128:(i + 1) * 128] for i in range(n // 128)]
        while len(parts) > 1:
            parts = [parts[i] + parts[i + 1]
                     for i in range(0, len(parts), 2)]
        return jnp.sum(parts[0].astype(jnp.float32), axis=1, keepdims=True)

    def body_a(i, cand_u):
        trial_u = cand_u | jnp.left_shift(jnp.int32(1), 15 - i)
        trial_s = trial_u - np.int32(32768)
        pat16 = trial_s ^ ((trial_s >> 15) & np.int32(0x7FFF))
        tb32 = jax.lax.bitcast_convert_type(
            jnp.left_shift(pat16, 16), jnp.float32)
        cnt = count_bf(tb32.astype(jnp.bfloat16))
        return jnp.where(cnt >= kf, trial_u, cand_u)

    cand_u = jax.lax.fori_loop(
        0, 16, body_a, jnp.zeros((bm, 1), jnp.int32))

    cs = cand_u - np.int32(32768)
    k32 = _f32_key(jnp.left_shift(cs ^ ((cs >> 15) & np.int32(0x7FFF)), 16))

    half = np.int32(32769)

    def body_b(i, lo_hi):
        lo, hi = lo_hi
        mid = lo + ((hi - lo) >> 1)
        tf = jax.lax.bitcast_convert_type(_f32_key(mid), jnp.float32)
        cnt = jnp.sum((a1 >= tf).astype(jnp.float32), axis=1, keepdims=True)
        ok = cnt >= kf
        return jnp.where(ok, mid, lo), jnp.where(ok, hi, mid)

    lo, _ = jax.lax.fori_loop(0, 9, body_b, (k32 - half, k32 + half))
    thr = jax.lax.bitcast_convert_type(_f32_key(lo), jnp.float32)

    a1s_ref[...] = jnp.where(a1 >= thr, a1, 0.0).astype(jnp.bfloat16)


def _decode_kernel(a1s_ref, w_ref, bd_ref, out_ref):
    kb = pl.program_id(1)
    part = jax.lax.dot_general(
        a1s_ref[...], w_ref[...], (((1,), (0,)), ((), ())),
        preferred_element_type=jnp.float32)

    @pl.when(kb == 0)
    def _init_out():
        out_ref[...] = bd_ref[...] + part

    @pl.when(kb > 0)
    def _acc_out():
        out_ref[...] += part


def kernel(x, W_enc, b_enc, b_dec, epoch):
    if x.ndim == 1:
        x = x[None, :]
    batch, in_dim = x.shape
    bn_dim = W_enc.shape[0]
    k_count = max(1, int(bn_dim * _A * _K_FRAC))

    x_bf = x.astype(jnp.bfloat16)
    w_bf = W_enc.astype(jnp.bfloat16)
    wt_bf = w_bf.T
    be2 = b_enc.reshape(1, -1).astype(jnp.float32)
    bd2 = b_dec.reshape(1, -1).astype(jnp.float32)

    bm_e, bn_e = min(512, batch), min(2048, bn_dim)
    a1 = pl.pallas_call(
        _encode_kernel,
        grid=(bn_dim // bn_e, batch // bm_e),
        in_specs=[
            pl.BlockSpec((bm_e, in_dim), lambda bn, bm: (bm, 0)),
            pl.BlockSpec((in_dim, bn_e), lambda bn, bm: (0, bn)),
            pl.BlockSpec((1, bn_e), lambda bn, bm: (0, bn)),
        ],
        out_specs=pl.BlockSpec((bm_e, bn_e), lambda bn, bm: (bm, bn)),
        out_shape=jax.ShapeDtypeStruct((batch, bn_dim), jnp.float32),
    )(x_bf, wt_bf, be2)

    bm_t = min(256, batch)
    a1s = pl.pallas_call(
        functools.partial(_select_kernel, k_count=k_count),
        grid=(batch // bm_t,),
        in_specs=[pl.BlockSpec((bm_t, bn_dim), lambda bm: (bm, 0))],
        out_specs=pl.BlockSpec((bm_t, bn_dim), lambda bm: (bm, 0)),
        out_shape=jax.ShapeDtypeStruct((batch, bn_dim), jnp.bfloat16),
    )(a1)

    bm_d, bk_d = min(1024, batch), min(2048, bn_dim)
    z2 = pl.pallas_call(
        _decode_kernel,
        grid=(batch // bm_d, bn_dim // bk_d),
        in_specs=[
            pl.BlockSpec((bm_d, bk_d), lambda bm, kb: (bm, kb)),
            pl.BlockSpec((bk_d, in_dim), lambda bm, kb: (kb, 0)),
            pl.BlockSpec((1, in_dim), lambda bm, kb: (0, 0)),
        ],
        out_specs=pl.BlockSpec((bm_d, in_dim), lambda bm, kb: (bm, 0)),
        out_shape=jax.ShapeDtypeStruct((batch, in_dim), jnp.float32),
    )(a1s, w_bf, bd2)

    return z2
